# Initial kernel scaffold; baseline (speedup 1.0000x reference)
import jax
import jax.numpy as jnp
from jax.experimental import pallas as pl


def kernel(token_ids, projection):
    raise NotImplementedError("write your pallas kernel here")


# speedup vs baseline: 204.0531x; 204.0531x over previous
import functools

import jax
import jax.numpy as jnp
from jax import lax
from jax.experimental import pallas as pl
from jax.experimental.pallas import tpu as pltpu
from jax.experimental.pallas import tpu_sc as plsc

VOCAB = 1_000_000
ROWS, COLS = 16384, 200
TOTAL = ROWS * COLS
NC, NS, L = 2, 16, 16
NW = NC * NS
PER_W = TOTAL // NW
CHUNK = 25_600
N_CHUNKS = PER_W // CHUNK
STAGE_TILES = 8
STAGE_SZ = VOCAB // STAGE_TILES
STAGE_SUB = 25_000


def _compressor(ids_hbm, proj_hbm, out_hbm, idx_v, val_v, table_sh, sem):
    cid = lax.axis_index("c")
    sid = lax.axis_index("s")
    wid = sid * NC + cid

    @pl.when(sid < STAGE_TILES)
    def _stage():
        for j in range(STAGE_SZ // STAGE_SUB):
            off = sid * STAGE_SZ + j * STAGE_SUB
            pltpu.sync_copy(proj_hbm.at[pl.ds(off, STAGE_SUB)],
                            idx_v.at[pl.ds(0, STAGE_SUB)])
            pltpu.sync_copy(idx_v.at[pl.ds(0, STAGE_SUB)],
                            table_sh.at[pl.ds(off, STAGE_SUB)])

    plsc.subcore_barrier()

    base0 = wid * PER_W
    for k in range(N_CHUNKS):
        base = base0 + k * CHUNK

        pltpu.sync_copy(ids_hbm.at[pl.ds(base, CHUNK)], idx_v)

        @plsc.parallel_loop(0, CHUNK, L, unroll=8)
        def _clamp(i):
            x = idx_v[pl.ds(i, L)]
            idx_v[pl.ds(i, L)] = jnp.minimum(jnp.maximum(x, 0), VOCAB - 1)

        pltpu.async_copy(table_sh.at[idx_v], val_v, sem).wait()

        pltpu.sync_copy(val_v, out_hbm.at[pl.ds(base, CHUNK)])


@jax.jit
def _run(ids_flat, projection):
    mesh = plsc.VectorSubcoreMesh(core_axis_name="c", subcore_axis_name="s")
    return pl.kernel(
        _compressor,
        out_type=pltpu.HBM((TOTAL,), jnp.int32),
        mesh=mesh,
        scratch_types=[
            pltpu.VMEM((CHUNK,), jnp.int32),
            pltpu.VMEM((CHUNK,), jnp.int32),
            pltpu.VMEM_SHARED((VOCAB,), jnp.int32),
            pltpu.SemaphoreType.DMA,
        ],
    )(ids_flat, projection)


def kernel(token_ids, projection):
    ids_flat = token_ids.reshape(TOTAL).astype(jnp.int32)
    out = _run(ids_flat, projection.astype(jnp.int32))
    return out.reshape(ROWS, COLS)


# speedup vs baseline: 228.7765x; 1.1212x over previous
import jax
import jax.numpy as jnp
from jax import lax
from jax.experimental import pallas as pl
from jax.experimental.pallas import tpu as pltpu
from jax.experimental.pallas import tpu_sc as plsc

VOCAB = 1_000_000
ROWS, COLS = 16384, 200
NC, NS, L = 2, 16, 16
NW = NC * NS
ROWS_W = ROWS // NW
CR = 64
CHUNK = CR * COLS
N_CHUNKS = ROWS_W // CR
STAGE_SUB = 10_000
N_SUB = VOCAB // STAGE_SUB
_CLAMP_COLS = tuple(range(0, COLS - L + 1, L)) + (COLS - L,)


def _body(ids_hbm, proj_hbm, out_hbm, buf2d_v, idx_v, val_v, stage_v,
          table_sh, sem):
    cid = lax.axis_index("c")
    sid = lax.axis_index("s")
    wid = sid * NC + cid

    for jj in range((N_SUB + NS - 1) // NS):
        j = sid + jj * NS

        @pl.when(j < N_SUB)
        def _stage():
            off = j * STAGE_SUB
            pltpu.sync_copy(proj_hbm.at[pl.ds(off, STAGE_SUB)], stage_v)
            pltpu.sync_copy(stage_v, table_sh.at[pl.ds(off, STAGE_SUB)])

    plsc.subcore_barrier()

    row0 = wid * ROWS_W
    for k in range(N_CHUNKS):
        base = row0 + k * CR

        pltpu.sync_copy(ids_hbm.at[pl.ds(base, CR)], buf2d_v)

        @plsc.parallel_loop(0, CR, 1, unroll=2)
        def _clamp(i):
            for c in _CLAMP_COLS:
                x = buf2d_v[i, pl.ds(c, L)]
                idx_v[pl.ds(i * COLS + c, L)] = jnp.minimum(
                    jnp.maximum(x, 0), VOCAB - 1)

        pltpu.async_copy(table_sh.at[idx_v], val_v, sem).wait()

        pltpu.sync_copy(val_v, out_hbm.at[pl.ds(base * COLS, CHUNK)])


@jax.jit
def _run(token_ids, projection):
    mesh = plsc.VectorSubcoreMesh(core_axis_name="c", subcore_axis_name="s")
    return pl.kernel(
        _body,
        out_type=pltpu.HBM((ROWS * COLS,), jnp.int32),
        mesh=mesh,
        scratch_types=[
            pltpu.VMEM((CR, COLS), jnp.int32),
            pltpu.VMEM((CHUNK,), jnp.int32),
            pltpu.VMEM((CHUNK,), jnp.int32),
            pltpu.VMEM((STAGE_SUB,), jnp.int32),
            pltpu.VMEM_SHARED((VOCAB,), jnp.int32),
            pltpu.SemaphoreType.DMA,
        ],
    )(token_ids, projection)


def kernel(token_ids, projection):
    out = _run(token_ids.astype(jnp.int32), projection.astype(jnp.int32))
    return out.reshape(ROWS, COLS)


# speedup vs baseline: 265.2791x; 1.1596x over previous
import jax
import jax.numpy as jnp
from jax import lax
from jax.experimental import pallas as pl
from jax.experimental.pallas import tpu as pltpu
from jax.experimental.pallas import tpu_sc as plsc

VOCAB = 1_000_000
ROWS, COLS = 16384, 200
TOTAL = ROWS * COLS
NC, NS, L = 2, 16, 16
NW = NC * NS
ROWS_W = ROWS // NW
CR = 64
CHUNK = CR * COLS
N_CHUNKS = ROWS_W // CR
STAGE_SUB = 10_000
N_SUB = VOCAB // STAGE_SUB
_CLAMP_COLS = tuple(range(0, COLS - L + 1, L)) + (COLS - L,)


def _body(ids_hbm, proj_hbm, out_hbm, buf2d_v, idx0_v, idx1_v, val0_v,
          val1_v, table_sh, sem, gsem):
    cid = lax.axis_index("c")
    sid = lax.axis_index("s")
    wid = sid * NC + cid
    idx_bufs = (idx0_v, idx1_v)
    val_bufs = (val0_v, val1_v)
    row0 = wid * ROWS_W

    def load_and_flatten(k, idx_v):
        pltpu.sync_copy(ids_hbm.at[pl.ds(row0 + k * CR, CR)], buf2d_v)

        @plsc.parallel_loop(0, CR, 1, unroll=2)
        def _clamp(i):
            for c in _CLAMP_COLS:
                x = buf2d_v[i, pl.ds(c, L)]
                idx_v[pl.ds(i * COLS + c, L)] = jnp.minimum(
                    jnp.maximum(x, 0), VOCAB - 1)

    load_and_flatten(0, idx_bufs[0])

    for jj in range((N_SUB + NS - 1) // NS):
        j = sid + jj * NS

        @pl.when(j < N_SUB)
        def _stage():
            off = j * STAGE_SUB
            bounce = val0_v.at[pl.ds(0, STAGE_SUB)]
            pltpu.sync_copy(proj_hbm.at[pl.ds(off, STAGE_SUB)], bounce)
            pltpu.sync_copy(bounce, table_sh.at[pl.ds(off, STAGE_SUB)])

    plsc.subcore_barrier()

    for k in range(N_CHUNKS):
        idx_v = idx_bufs[k % 2]
        val_v = val_bufs[k % 2]

        if k >= 2:
            pltpu.make_async_copy(proj_hbm.at[pl.ds(0, CHUNK)], val_v,
                                  sem).wait()

        gather = pltpu.async_copy(table_sh.at[idx_v], val_v, gsem)

        if k + 1 < N_CHUNKS:
            load_and_flatten(k + 1, idx_bufs[(k + 1) % 2])

        gather.wait()

        pltpu.async_copy(val_v, out_hbm.at[pl.ds((row0 + k * CR) * COLS,
                                                 CHUNK)], sem)

    for k in (N_CHUNKS - 2, N_CHUNKS - 1):
        pltpu.make_async_copy(proj_hbm.at[pl.ds(0, CHUNK)],
                              val_bufs[k % 2], sem).wait()


@jax.jit
def _run(token_ids, projection):
    mesh = plsc.VectorSubcoreMesh(core_axis_name="c", subcore_axis_name="s")
    return pl.kernel(
        _body,
        out_type=pltpu.HBM((TOTAL,), jnp.int32),
        mesh=mesh,
        scratch_types=[
            pltpu.VMEM((CR, COLS), jnp.int32),
            pltpu.VMEM((CHUNK,), jnp.int32),
            pltpu.VMEM((CHUNK,), jnp.int32),
            pltpu.VMEM((CHUNK,), jnp.int32),
            pltpu.VMEM((CHUNK,), jnp.int32),
            pltpu.VMEM_SHARED((VOCAB,), jnp.int32),
            pltpu.SemaphoreType.DMA,
            pltpu.SemaphoreType.DMA,
        ],
    )(token_ids, projection)


def kernel(token_ids, projection):
    out = _run(token_ids.astype(jnp.int32), projection.astype(jnp.int32))
    return out.reshape(ROWS, COLS)


# speedup vs baseline: 378.9613x; 1.4285x over previous
import jax
import jax.numpy as jnp
from jax import lax
from jax.experimental import pallas as pl
from jax.experimental.pallas import tpu as pltpu
from jax.experimental.pallas import tpu_sc as plsc

VOCAB = 1_000_000
ROWS, COLS = 16384, 200
NC, NS, L = 2, 16, 16
NW = NC * NS
CH_R, CH_C = 8, 512
CHUNK = CH_R * CH_C
CB = COLS // CH_R
WB = ROWS // CH_C
N_CHUNKS = CB * WB // NW
STAGE_SUB = 20_000
N_SUB = VOCAB // STAGE_SUB


def _body(ids_hbm, proj_hbm, out_hbm, in2d_v, out2d_v, idx0_v, idx1_v,
          val0_v, val1_v, stage_v, table_sh, sem, gsem):
    cid = lax.axis_index("c")
    sid = lax.axis_index("s")
    wid = sid * NC + cid
    idx_bufs = (idx0_v, idx1_v)
    val_bufs = (val0_v, val1_v)

    def chunk_slices(k):
        m = wid * N_CHUNKS + k
        a, cb = m // WB, m % WB
        return pl.ds(a * CH_R, CH_R), pl.ds(cb * CH_C, CH_C)

    def load_and_flatten(k, idx_v):
        ra, ca = chunk_slices(k)
        pltpu.sync_copy(ids_hbm.at[ra, ca], in2d_v)

        @plsc.parallel_loop(0, CH_R, 1, unroll=2)
        def _clamp(i):
            for c in range(0, CH_C, L):
                x = in2d_v[i, pl.ds(c, L)]
                idx_v[pl.ds(i * CH_C + c, L)] = jnp.minimum(
                    jnp.maximum(x, 0), VOCAB - 1)

    load_and_flatten(0, idx_bufs[0])

    for jj in range((N_SUB + NS - 1) // NS):
        j = sid + jj * NS

        @pl.when(j < N_SUB)
        def _stage():
            off = j * STAGE_SUB
            pltpu.sync_copy(proj_hbm.at[pl.ds(off, STAGE_SUB)], stage_v)
            pltpu.sync_copy(stage_v, table_sh.at[pl.ds(off, STAGE_SUB)])

    plsc.subcore_barrier()

    def chunk_body(k, parity, prefetch):
        idx_v = idx_bufs[parity]
        val_v = val_bufs[parity]

        gather = pltpu.async_copy(table_sh.at[idx_v], val_v, gsem)

        if prefetch:
            load_and_flatten(k + 1, idx_bufs[1 - parity])

        gather.wait()

        @plsc.parallel_loop(0, CH_R, 1, unroll=2)
        def _unflatten(i):
            for c in range(0, CH_C, L):
                out2d_v[i, pl.ds(c, L)] = val_v[pl.ds(i * CH_C + c, L)]

        ra, ca = chunk_slices(k)
        pltpu.sync_copy(out2d_v, out_hbm.at[ra, ca])

    @pl.loop(0, N_CHUNKS - 1, step=2)
    def _pair(k):
        chunk_body(k, 0, True)
        chunk_body(k + 1, 1, True)

    chunk_body(N_CHUNKS - 1, 0, False)


@jax.jit
def _run(ids_t, projection):
    mesh = plsc.VectorSubcoreMesh(core_axis_name="c", subcore_axis_name="s")
    return pl.kernel(
        _body,
        out_type=pltpu.HBM((COLS, ROWS), jnp.int32),
        mesh=mesh,
        scratch_types=[
            pltpu.VMEM((CH_R, CH_C), jnp.int32),
            pltpu.VMEM((CH_R, CH_C), jnp.int32),
            pltpu.VMEM((CHUNK,), jnp.int32),
            pltpu.VMEM((CHUNK,), jnp.int32),
            pltpu.VMEM((CHUNK,), jnp.int32),
            pltpu.VMEM((CHUNK,), jnp.int32),
            pltpu.VMEM((STAGE_SUB,), jnp.int32),
            pltpu.VMEM_SHARED((VOCAB,), jnp.int32),
            pltpu.SemaphoreType.DMA,
            pltpu.SemaphoreType.DMA,
        ],
    )(ids_t, projection)


def kernel(token_ids, projection):
    ids_t = token_ids.astype(jnp.int32).T
    out_t = _run(ids_t, projection.astype(jnp.int32))
    return out_t.T


# speedup vs baseline: 428.7778x; 1.1315x over previous
import jax
import jax.numpy as jnp
from jax import lax
from jax.experimental import pallas as pl
from jax.experimental.pallas import tpu as pltpu
from jax.experimental.pallas import tpu_sc as plsc

VOCAB = 1_000_000
ROWS, COLS = 16384, 200
NC, NS, L = 2, 16, 16
NW = NC * NS
CH_R, CH_C = 8, 512
CHUNK = CH_R * CH_C
WB = ROWS // CH_C
N_CHUNKS = (COLS // CH_R) * WB // NW
STAGE_SUB = 10_000
N_SUB = VOCAB // STAGE_SUB
MAX_JJ = (N_SUB + NS - 1) // NS


def _body(ids_hbm, proj_hbm, out_hbm, in2d_v, out2d0_v, out2d1_v, idx0_v,
          idx1_v, val0_v, val1_v, stg0_v, stg1_v, table_sh, gsem, lsem,
          ssem):
    cid = lax.axis_index("c")
    sid = lax.axis_index("s")
    wid = sid * NC + cid
    idx_bufs = (idx0_v, idx1_v)
    val_bufs = (val0_v, val1_v)
    out_bufs = (out2d0_v, out2d1_v)
    stg_bufs = (stg0_v, stg1_v)

    def chunk_slices(k):
        m = wid * N_CHUNKS + k
        return pl.ds((m // WB) * CH_R, CH_R), pl.ds((m % WB) * CH_C, CH_C)

    def load_fire(k):
        ra, ca = chunk_slices(k)
        return pltpu.async_copy(ids_hbm.at[ra, ca], in2d_v, lsem)

    def flatten(parity):
        idx_v = idx_bufs[parity]

        @plsc.parallel_loop(0, CH_R, 1, unroll=2)
        def _clamp(i):
            for c in range(0, CH_C, L):
                x = in2d_v[i, pl.ds(c, L)]
                idx_v[pl.ds(i * CH_C + c, L)] = jnp.minimum(
                    jnp.maximum(x, 0), VOCAB - 1)

    def unflatten(parity):
        val_v = val_bufs[parity]
        out_v = out_bufs[parity]

        @plsc.parallel_loop(0, CH_R, 1, unroll=2)
        def _un(i):
            for c in range(0, CH_C, L):
                out_v[i, pl.ds(c, L)] = val_v[pl.ds(i * CH_C + c, L)]

    def gather_fire(parity):
        pltpu.async_copy(table_sh.at[idx_bufs[parity]], val_bufs[parity],
                         gsem)

    def gather_wait(parity):
        pltpu.make_async_copy(proj_hbm.at[pl.ds(0, CHUNK)],
                              val_bufs[parity], gsem).wait()

    def store_fire(k, parity):
        ra, ca = chunk_slices(k)
        pltpu.async_copy(out_bufs[parity], out_hbm.at[ra, ca], ssem)

    def store_drain(parity):
        pltpu.make_async_copy(ids_hbm.at[pl.ds(0, CH_R), pl.ds(0, CH_C)],
                              out_bufs[parity], ssem).wait()

    ld0 = load_fire(0)

    for jj in range(MAX_JJ):
        j = sid + jj * NS

        @pl.when(j < N_SUB)
        def _stage():
            h = jj % 2
            off = j * STAGE_SUB
            if jj >= 2:
                pltpu.make_async_copy(proj_hbm.at[pl.ds(0, STAGE_SUB)],
                                      stg_bufs[h], ssem).wait()
            pltpu.async_copy(proj_hbm.at[pl.ds(off, STAGE_SUB)],
                             stg_bufs[h], lsem).wait()
            pltpu.async_copy(stg_bufs[h], table_sh.at[pl.ds(off, STAGE_SUB)],
                             ssem)

    for _ in range(2):
        pltpu.make_async_copy(proj_hbm.at[pl.ds(0, STAGE_SUB)],
                              stg_bufs[0], ssem).wait()

    ld0.wait()
    flatten(0)
    plsc.subcore_barrier()

    gather_fire(0)
    ld1 = load_fire(1)
    ld1.wait()
    flatten(1)

    def steady(k, parity, guard):
        gather_fire(parity)
        if guard:
            @pl.when(k + 1 < N_CHUNKS)
            def _pf_fire():
                load_fire(k + 1)
        else:
            load_fire(k + 1)
        gather_wait(1 - parity)
        if guard:
            @pl.when(k >= 3)
            def _drain():
                store_drain(1 - parity)
        elif k >= 3:
            store_drain(1 - parity)
        unflatten(1 - parity)
        store_fire(k - 1, 1 - parity)
        if guard:
            @pl.when(k + 1 < N_CHUNKS)
            def _pf_flat():
                pltpu.make_async_copy(
                    ids_hbm.at[pl.ds(0, CH_R), pl.ds(0, CH_C)], in2d_v,
                    lsem).wait()
                flatten(1 - parity)
        else:
            pltpu.make_async_copy(
                ids_hbm.at[pl.ds(0, CH_R), pl.ds(0, CH_C)], in2d_v,
                lsem).wait()
            flatten(1 - parity)

    steady(1, 1, False)
    steady(2, 0, False)

    @pl.loop(3, N_CHUNKS, step=2)
    def _pair(k):
        steady(k, 1, True)
        steady(k + 1, 0, True)

    gather_wait(0)
    store_drain(0)
    unflatten(0)
    store_fire(N_CHUNKS - 1, 0)
    store_drain(1)
    store_drain(0)


@jax.jit
def _run(ids_t, projection):
    mesh = plsc.VectorSubcoreMesh(core_axis_name="c", subcore_axis_name="s")
    return pl.kernel(
        _body,
        out_type=pltpu.HBM((COLS, ROWS), jnp.int32),
        mesh=mesh,
        scratch_types=[
            pltpu.VMEM((CH_R, CH_C), jnp.int32),
            pltpu.VMEM((CH_R, CH_C), jnp.int32),
            pltpu.VMEM((CH_R, CH_C), jnp.int32),
            pltpu.VMEM((CHUNK,), jnp.int32),
            pltpu.VMEM((CHUNK,), jnp.int32),
            pltpu.VMEM((CHUNK,), jnp.int32),
            pltpu.VMEM((CHUNK,), jnp.int32),
            pltpu.VMEM((STAGE_SUB,), jnp.int32),
            pltpu.VMEM((STAGE_SUB,), jnp.int32),
            pltpu.VMEM_SHARED((VOCAB,), jnp.int32),
            pltpu.SemaphoreType.DMA,
            pltpu.SemaphoreType.DMA,
            pltpu.SemaphoreType.DMA,
        ],
    )(ids_t, projection)


def kernel(token_ids, projection):
    ids_t = token_ids.astype(jnp.int32).T
    out_t = _run(ids_t, projection.astype(jnp.int32))
    return out_t.T


# speedup vs baseline: 519.8187x; 1.2123x over previous
import jax
import jax.numpy as jnp
from jax import lax
from jax.experimental import pallas as pl
from jax.experimental.pallas import tpu as pltpu
from jax.experimental.pallas import tpu_sc as plsc

VOCAB = 1_000_000
ROWS, COLS = 16384, 200
NC, NS, L = 2, 16, 16
NW = NC * NS
CH_R, CH_C = 8, 512
CHUNK = CH_R * CH_C
WB = ROWS // CH_C
N_CHUNKS = (COLS // CH_R) * WB // NW
STAGE_SUB = 10_000
N_SUB = VOCAB // STAGE_SUB
MAX_JJ = (N_SUB + NS - 1) // NS


def _body(ids_hbm, proj_hbm, out_hbm, in2d_v, out2d0_v, out2d1_v, idx0_v,
          idx1_v, val0_v, val1_v, stg0_v, stg1_v, table_sh, gsem, lsem,
          ssem):
    cid = lax.axis_index("c")
    sid = lax.axis_index("s")
    wid = sid * NC + cid
    idx_bufs = (idx0_v, idx1_v)
    val_bufs = (val0_v, val1_v)
    out_bufs = (out2d0_v, out2d1_v)
    stg_bufs = (stg0_v, stg1_v)

    def chunk_slices(k):
        m = wid * N_CHUNKS + k
        return pl.ds((m // WB) * CH_R, CH_R), pl.ds((m % WB) * CH_C, CH_C)

    def load_fire(k):
        ra, ca = chunk_slices(k)
        return pltpu.async_copy(ids_hbm.at[ra, ca], in2d_v, lsem)

    def flatten(parity):
        idx_v = idx_bufs[parity]

        @plsc.parallel_loop(0, CHUNK, L, unroll=4)
        def _clamp(p):
            x = in2d_v[p // CH_C, pl.ds(p % CH_C, L)]
            idx_v[pl.ds(p, L)] = jnp.minimum(jnp.maximum(x, 0), VOCAB - 1)

    def unflatten(parity):
        val_v = val_bufs[parity]
        out_v = out_bufs[parity]

        @plsc.parallel_loop(0, CHUNK, L, unroll=4)
        def _un(p):
            out_v[p // CH_C, pl.ds(p % CH_C, L)] = val_v[pl.ds(p, L)]

    def gather_fire(parity):
        pltpu.async_copy(table_sh.at[idx_bufs[parity]], val_bufs[parity],
                         gsem)

    def gather_wait(parity):
        pltpu.make_async_copy(proj_hbm.at[pl.ds(0, CHUNK)],
                              val_bufs[parity], gsem).wait()

    def store_fire(k, parity):
        ra, ca = chunk_slices(k)
        pltpu.async_copy(out_bufs[parity], out_hbm.at[ra, ca], ssem)

    def store_drain(parity):
        pltpu.make_async_copy(ids_hbm.at[pl.ds(0, CH_R), pl.ds(0, CH_C)],
                              out_bufs[parity], ssem).wait()

    ld0 = load_fire(0)

    for jj in range(MAX_JJ):
        j = sid + jj * NS

        @pl.when(j < N_SUB)
        def _stage():
            h = jj % 2
            off = j * STAGE_SUB
            if jj >= 2:
                pltpu.make_async_copy(proj_hbm.at[pl.ds(0, STAGE_SUB)],
                                      stg_bufs[h], ssem).wait()
            pltpu.async_copy(proj_hbm.at[pl.ds(off, STAGE_SUB)],
                             stg_bufs[h], lsem).wait()
            pltpu.async_copy(stg_bufs[h], table_sh.at[pl.ds(off, STAGE_SUB)],
                             ssem)

    for _ in range(2):
        pltpu.make_async_copy(proj_hbm.at[pl.ds(0, STAGE_SUB)],
                              stg_bufs[0], ssem).wait()

    ld0.wait()
    flatten(0)
    plsc.subcore_barrier()

    gather_fire(0)
    ld1 = load_fire(1)
    ld1.wait()
    flatten(1)

    def steady(k, parity, guard):
        gather_fire(parity)
        if guard:
            @pl.when(k + 1 < N_CHUNKS)
            def _pf_fire():
                load_fire(k + 1)
        else:
            load_fire(k + 1)
        gather_wait(1 - parity)
        if guard:
            @pl.when(k >= 3)
            def _drain():
                store_drain(1 - parity)
        elif k >= 3:
            store_drain(1 - parity)
        unflatten(1 - parity)
        store_fire(k - 1, 1 - parity)
        if guard:
            @pl.when(k + 1 < N_CHUNKS)
            def _pf_flat():
                pltpu.make_async_copy(
                    ids_hbm.at[pl.ds(0, CH_R), pl.ds(0, CH_C)], in2d_v,
                    lsem).wait()
                flatten(1 - parity)
        else:
            pltpu.make_async_copy(
                ids_hbm.at[pl.ds(0, CH_R), pl.ds(0, CH_C)], in2d_v,
                lsem).wait()
            flatten(1 - parity)

    steady(1, 1, False)
    steady(2, 0, False)

    @pl.loop(3, N_CHUNKS, step=2)
    def _pair(k):
        steady(k, 1, True)
        steady(k + 1, 0, True)

    gather_wait(0)
    store_drain(0)
    unflatten(0)
    store_fire(N_CHUNKS - 1, 0)
    store_drain(1)
    store_drain(0)


@jax.jit
def _run(ids_t, projection):
    mesh = plsc.VectorSubcoreMesh(core_axis_name="c", subcore_axis_name="s")
    return pl.kernel(
        _body,
        out_type=pltpu.HBM((COLS, ROWS), jnp.int32),
        mesh=mesh,
        scratch_types=[
            pltpu.VMEM((CH_R, CH_C), jnp.int32),
            pltpu.VMEM((CH_R, CH_C), jnp.int32),
            pltpu.VMEM((CH_R, CH_C), jnp.int32),
            pltpu.VMEM((CHUNK,), jnp.int32),
            pltpu.VMEM((CHUNK,), jnp.int32),
            pltpu.VMEM((CHUNK,), jnp.int32),
            pltpu.VMEM((CHUNK,), jnp.int32),
            pltpu.VMEM((STAGE_SUB,), jnp.int32),
            pltpu.VMEM((STAGE_SUB,), jnp.int32),
            pltpu.VMEM_SHARED((VOCAB,), jnp.int32),
            pltpu.SemaphoreType.DMA,
            pltpu.SemaphoreType.DMA,
            pltpu.SemaphoreType.DMA,
        ],
    )(ids_t, projection)


def kernel(token_ids, projection):
    ids_t = token_ids.astype(jnp.int32).T
    out_t = _run(ids_t, projection.astype(jnp.int32))
    return out_t.T


# speedup vs baseline: 522.2330x; 1.0046x over previous
import jax
import jax.numpy as jnp
from jax import lax
from jax.experimental import pallas as pl
from jax.experimental.pallas import tpu as pltpu
from jax.experimental.pallas import tpu_sc as plsc

VOCAB = 1_000_000
ROWS, COLS = 16384, 200
NC, NS, L = 2, 16, 16
NW = NC * NS
CH_R, CH_C = 8, 512
CHUNK = CH_R * CH_C
WB = ROWS // CH_C
N_CHUNKS = (COLS // CH_R) * WB // NW
STAGE_SUB = 10_000
N_SUB = VOCAB // STAGE_SUB
MAX_JJ = (N_SUB + NS - 1) // NS


def _body(ids_hbm, proj_hbm, out_hbm, in2d_v, out2d0_v, out2d1_v, idx0_v,
          idx1_v, val0_v, val1_v, stg0_v, stg1_v, table_sh, gsem, lsem,
          ssem):
    cid = lax.axis_index("c")
    sid = lax.axis_index("s")
    wid = sid * NC + cid
    idx_bufs = (idx0_v, idx1_v)
    val_bufs = (val0_v, val1_v)
    out_bufs = (out2d0_v, out2d1_v)
    stg_bufs = (stg0_v, stg1_v)

    def chunk_slices(k):
        m = wid * N_CHUNKS + k
        return pl.ds((m // WB) * CH_R, CH_R), pl.ds((m % WB) * CH_C, CH_C)

    def load_fire(k):
        ra, ca = chunk_slices(k)
        return pltpu.async_copy(ids_hbm.at[ra, ca], in2d_v, lsem)

    def flatten(parity):
        idx_v = idx_bufs[parity]

        @plsc.parallel_loop(0, CHUNK, L, unroll=4)
        def _clamp(p):
            x = in2d_v[p // CH_C, pl.ds(p % CH_C, L)]
            idx_v[pl.ds(p, L)] = jnp.minimum(jnp.maximum(x, 0), VOCAB - 1)

    def unflatten(parity):
        val_v = val_bufs[parity]
        out_v = out_bufs[parity]

        @plsc.parallel_loop(0, CHUNK, L, unroll=4)
        def _un(p):
            out_v[p // CH_C, pl.ds(p % CH_C, L)] = val_v[pl.ds(p, L)]

    def gather_fire(parity):
        pltpu.async_copy(table_sh.at[idx_bufs[parity]], val_bufs[parity],
                         gsem)

    def gather_wait(parity):
        pltpu.make_async_copy(proj_hbm.at[pl.ds(0, CHUNK)],
                              val_bufs[parity], gsem).wait()

    def store_fire(k, parity):
        ra, ca = chunk_slices(k)
        pltpu.async_copy(out_bufs[parity], out_hbm.at[ra, ca], ssem)

    def store_drain(parity):
        pltpu.make_async_copy(ids_hbm.at[pl.ds(0, CH_R), pl.ds(0, CH_C)],
                              out_bufs[parity], ssem).wait()

    ld0 = load_fire(0)

    for jj in range(MAX_JJ):
        j = sid + jj * NS

        @pl.when(j < N_SUB)
        def _stage():
            h = jj % 2
            off = j * STAGE_SUB
            if jj >= 2:
                pltpu.make_async_copy(proj_hbm.at[pl.ds(0, STAGE_SUB)],
                                      stg_bufs[h], ssem).wait()
            pltpu.async_copy(proj_hbm.at[pl.ds(off, STAGE_SUB)],
                             stg_bufs[h], lsem).wait()
            pltpu.async_copy(stg_bufs[h], table_sh.at[pl.ds(off, STAGE_SUB)],
                             ssem)

    for _ in range(2):
        pltpu.make_async_copy(proj_hbm.at[pl.ds(0, STAGE_SUB)],
                              stg_bufs[0], ssem).wait()

    ld0.wait()
    flatten(0)
    plsc.subcore_barrier()

    gather_fire(0)
    ld1 = load_fire(1)
    ld1.wait()
    flatten(1)

    def steady(k, parity):
        gather_fire(parity)

        @pl.when(k + 1 < N_CHUNKS)
        def _pf_fire():
            load_fire(k + 1)

        gather_wait(1 - parity)

        @pl.when(k >= 3)
        def _drain():
            store_drain(1 - parity)

        unflatten(1 - parity)
        store_fire(k - 1, 1 - parity)

        @pl.when(k + 1 < N_CHUNKS)
        def _pf_flat():
            pltpu.make_async_copy(
                ids_hbm.at[pl.ds(0, CH_R), pl.ds(0, CH_C)], in2d_v,
                lsem).wait()
            flatten(1 - parity)

    @pl.loop(1, N_CHUNKS, step=2)
    def _pair(k):
        steady(k, 1)
        steady(k + 1, 0)

    gather_wait(0)
    store_drain(0)
    unflatten(0)
    store_fire(N_CHUNKS - 1, 0)
    store_drain(1)
    store_drain(0)


@jax.jit
def _run(ids_t, projection):
    mesh = plsc.VectorSubcoreMesh(core_axis_name="c", subcore_axis_name="s")
    return pl.kernel(
        _body,
        out_type=pltpu.HBM((COLS, ROWS), jnp.int32),
        mesh=mesh,
        scratch_types=[
            pltpu.VMEM((CH_R, CH_C), jnp.int32),
            pltpu.VMEM((CH_R, CH_C), jnp.int32),
            pltpu.VMEM((CH_R, CH_C), jnp.int32),
            pltpu.VMEM((CHUNK,), jnp.int32),
            pltpu.VMEM((CHUNK,), jnp.int32),
            pltpu.VMEM((CHUNK,), jnp.int32),
            pltpu.VMEM((CHUNK,), jnp.int32),
            pltpu.VMEM((STAGE_SUB,), jnp.int32),
            pltpu.VMEM((STAGE_SUB,), jnp.int32),
            pltpu.VMEM_SHARED((VOCAB,), jnp.int32),
            pltpu.SemaphoreType.DMA,
            pltpu.SemaphoreType.DMA,
            pltpu.SemaphoreType.DMA,
        ],
    )(ids_t, projection)


def kernel(token_ids, projection):
    ids_t = token_ids.astype(jnp.int32).T
    out_t = _run(ids_t, projection.astype(jnp.int32))
    return out_t.T


# speedup vs baseline: 554.5834x; 1.0619x over previous
import jax
import jax.numpy as jnp
from jax import lax
from jax.experimental import pallas as pl
from jax.experimental.pallas import tpu as pltpu
from jax.experimental.pallas import tpu_sc as plsc

VOCAB = 1_000_000
ROWS, COLS = 16384, 200
NC, NS, L = 2, 16, 16
NW = NC * NS
CH_R, CH_C = 40, 128
CHUNK = CH_R * CH_C
WB = ROWS // CH_C
N_CHUNKS = (COLS // CH_R) * WB // NW
STAGE_SUB = 10_000
N_SUB = VOCAB // STAGE_SUB
MAX_JJ = (N_SUB + NS - 1) // NS


def _body(ids_hbm, proj_hbm, out_hbm, in2d_v, out2d0_v, out2d1_v, idx0_v,
          idx1_v, val0_v, val1_v, stg0_v, stg1_v, table_sh, gsem0, gsem1,
          lsem, ssem0, ssem1):
    cid = lax.axis_index("c")
    sid = lax.axis_index("s")
    wid = sid * NC + cid
    idx_bufs = (idx0_v, idx1_v)
    val_bufs = (val0_v, val1_v)
    out_bufs = (out2d0_v, out2d1_v)
    stg_bufs = (stg0_v, stg1_v)
    gsems = (gsem0, gsem1)
    ssems = (ssem0, ssem1)

    def chunk_slices(k):
        m = wid * N_CHUNKS + k
        return pl.ds((m // WB) * CH_R, CH_R), pl.ds((m % WB) * CH_C, CH_C)

    def load_fire(k):
        ra, ca = chunk_slices(k)
        return pltpu.async_copy(ids_hbm.at[ra, ca], in2d_v, lsem)

    def flatten(parity):
        idx_v = idx_bufs[parity]

        @plsc.parallel_loop(0, CHUNK, L, unroll=4)
        def _clamp(p):
            x = in2d_v[p // CH_C, pl.ds(p % CH_C, L)]
            idx_v[pl.ds(p, L)] = jnp.minimum(jnp.maximum(x, 0), VOCAB - 1)

    def unflatten(parity):
        val_v = val_bufs[parity]
        out_v = out_bufs[parity]

        @plsc.parallel_loop(0, CHUNK, L, unroll=4)
        def _un(p):
            out_v[p // CH_C, pl.ds(p % CH_C, L)] = val_v[pl.ds(p, L)]

    def gather_fire(parity):
        pltpu.async_copy(table_sh.at[idx_bufs[parity]], val_bufs[parity],
                         gsems[parity])

    def gather_wait(parity):
        pltpu.make_async_copy(proj_hbm.at[pl.ds(0, CHUNK)],
                              val_bufs[parity], gsems[parity]).wait()

    def store_fire(k, parity):
        ra, ca = chunk_slices(k)
        pltpu.async_copy(out_bufs[parity], out_hbm.at[ra, ca], ssems[parity])

    def store_drain(parity):
        pltpu.make_async_copy(ids_hbm.at[pl.ds(0, CH_R), pl.ds(0, CH_C)],
                              out_bufs[parity], ssems[parity]).wait()

    ld0 = load_fire(0)

    for jj in range(MAX_JJ):
        j = sid + jj * NS

        @pl.when(j < N_SUB)
        def _stage():
            h = jj % 2
            off = j * STAGE_SUB
            if jj >= 2:
                pltpu.make_async_copy(proj_hbm.at[pl.ds(0, STAGE_SUB)],
                                      stg_bufs[h], ssems[h]).wait()
            pltpu.async_copy(proj_hbm.at[pl.ds(off, STAGE_SUB)],
                             stg_bufs[h], lsem).wait()
            pltpu.async_copy(stg_bufs[h], table_sh.at[pl.ds(off, STAGE_SUB)],
                             ssems[h])

    for h in range(2):
        pltpu.make_async_copy(proj_hbm.at[pl.ds(0, STAGE_SUB)],
                              stg_bufs[h], ssems[h]).wait()

    ld0.wait()
    flatten(0)
    plsc.subcore_barrier()

    gather_fire(0)
    ld1 = load_fire(1)
    ld1.wait()
    flatten(1)

    def steady(k, parity):
        gather_fire(parity)

        @pl.when(k + 1 < N_CHUNKS)
        def _pf_fire():
            load_fire(k + 1)

        gather_wait(1 - parity)

        @pl.when(k >= 3)
        def _drain():
            store_drain(1 - parity)

        unflatten(1 - parity)
        store_fire(k - 1, 1 - parity)

        @pl.when(k + 1 < N_CHUNKS)
        def _pf_flat():
            pltpu.make_async_copy(
                ids_hbm.at[pl.ds(0, CH_R), pl.ds(0, CH_C)], in2d_v,
                lsem).wait()
            flatten(1 - parity)

    @pl.loop(1, N_CHUNKS - 1, step=2)
    def _pair(k):
        steady(k, 1)
        steady(k + 1, 0)

    steady(N_CHUNKS - 1, 1)

    gather_wait(1)
    store_drain(1)
    unflatten(1)
    store_fire(N_CHUNKS - 1, 1)
    store_drain(0)
    store_drain(1)


@jax.jit
def _run(ids_t, projection):
    mesh = plsc.VectorSubcoreMesh(core_axis_name="c", subcore_axis_name="s")
    return pl.kernel(
        _body,
        out_type=pltpu.HBM((COLS, ROWS), jnp.int32),
        mesh=mesh,
        scratch_types=[
            pltpu.VMEM((CH_R, CH_C), jnp.int32),
            pltpu.VMEM((CH_R, CH_C), jnp.int32),
            pltpu.VMEM((CH_R, CH_C), jnp.int32),
            pltpu.VMEM((CHUNK,), jnp.int32),
            pltpu.VMEM((CHUNK,), jnp.int32),
            pltpu.VMEM((CHUNK,), jnp.int32),
            pltpu.VMEM((CHUNK,), jnp.int32),
            pltpu.VMEM((STAGE_SUB,), jnp.int32),
            pltpu.VMEM((STAGE_SUB,), jnp.int32),
            pltpu.VMEM_SHARED((VOCAB,), jnp.int32),
            pltpu.SemaphoreType.DMA,
            pltpu.SemaphoreType.DMA,
            pltpu.SemaphoreType.DMA,
            pltpu.SemaphoreType.DMA,
            pltpu.SemaphoreType.DMA,
        ],
    )(ids_t, projection)


def kernel(token_ids, projection):
    ids_t = token_ids.astype(jnp.int32).T
    out_t = _run(ids_t, projection.astype(jnp.int32))
    return out_t.T
